# Initial kernel scaffold; baseline (speedup 1.0000x reference)
import jax
import jax.numpy as jnp
from jax.experimental import pallas as pl


def kernel(x, Wr, br, W1, b1, W2, b2):
    raise NotImplementedError("write your pallas kernel here")


# speedup vs baseline: 9.0958x; 9.0958x over previous
import functools

import jax
import jax.numpy as jnp
from jax import lax
from jax.experimental import pallas as pl
from jax.experimental.pallas import tpu as pltpu
from jax.experimental.pallas import tpu_sc as plsc

D = 768
E = 8
H = 3072
T = 2048
S = 2 * T
BT = 512
NB = 16
RPAD = NB * BT
CB = 1024
NCB = S // CB

SC_CORES = 2
SC_SUBCORES = 16
NW = SC_CORES * SC_SUBCORES
PPW = S // NW
_DCH = 32
_NDCH = PPW // _DCH
TPW = T // NW
_CH = 16
_NCH = TPW // _CH


def _router_body(x_ref, wr_ref, br_ref,
                 dest_ref, w0_ref, w1_ref, bg_ref, valid_ref,
                 oh_ref, rank_ref):
    x = x_ref[...]
    logits = lax.dot_general(x, wr_ref[...], (((1,), (1,)), ((), ())),
                             preferred_element_type=jnp.float32)
    logits = logits + br_ref[...]

    lane = lax.broadcasted_iota(jnp.int32, (T, E), 1)
    m1 = jnp.max(logits, axis=1, keepdims=True)
    i0 = jnp.min(jnp.where(logits == m1, lane, E), axis=1, keepdims=True)
    oh0 = (lane == i0)
    masked = jnp.where(oh0, -1e30, logits)
    m2 = jnp.max(masked, axis=1, keepdims=True)
    i1 = jnp.min(jnp.where(masked == m2, lane, E), axis=1, keepdims=True)
    oh1 = (lane == i1)

    a = jnp.exp(m2 - m1)
    w0_ref[...] = jnp.broadcast_to(1.0 / (1.0 + a), (T, 16))
    w1_ref[...] = jnp.broadcast_to(a / (1.0 + a), (T, 16))

    oh_ref[...] = jnp.concatenate(
        [oh0.astype(jnp.float32), oh1.astype(jnp.float32)], axis=0)

    r_i = lax.broadcasted_iota(jnp.int32, (CB, CB), 0)
    c_i = lax.broadcasted_iota(jnp.int32, (CB, CB), 1)
    tri = (r_i >= c_i).astype(jnp.float32)

    def csum_block(b, carry):
        off = pl.multiple_of(b * CB, CB)
        blk = oh_ref[pl.ds(off, CB), :]
        csum = lax.dot_general(tri, blk, (((1,), (0,)), ((), ())),
                               preferred_element_type=jnp.float32) + carry
        rank_ref[pl.ds(off, CB), :] = (
            jnp.sum(csum * blk, axis=1, keepdims=True) - 1.0)
        return carry + jnp.sum(blk, axis=0, keepdims=True)

    counts = lax.fori_loop(0, NCB, csum_block,
                           jnp.zeros((1, E), jnp.float32))

    pc = jnp.ceil(counts * (1.0 / BT)) * BT
    e_i = lax.broadcasted_iota(jnp.int32, (E, E), 0)
    e_j = lax.broadcasted_iota(jnp.int32, (E, E), 1)
    excl = (e_i < e_j).astype(jnp.float32)
    incl = (e_i <= e_j).astype(jnp.float32)
    offs = lax.dot_general(pc, excl, (((1,), (0,)), ((), ())),
                           preferred_element_type=jnp.float32)
    pcc = lax.dot_general(pc, incl, (((1,), (0,)), ((), ())),
                          preferred_element_type=jnp.float32)

    def dest_block(b, _):
        off = pl.multiple_of(b * CB, CB)
        blk = oh_ref[pl.ds(off, CB), :]
        base = jnp.sum(blk * offs, axis=1, keepdims=True)
        dest_ref[pl.ds(off, CB), :] = (
            base + rank_ref[pl.ds(off, CB), :]).astype(jnp.int32)
        return 0

    lax.fori_loop(0, NCB, dest_block, 0)

    brow = lax.broadcasted_iota(
        jnp.int32, (NB, 1), 0).astype(jnp.float32) * float(BT)
    bg = jnp.sum((brow >= pcc).astype(jnp.float32), axis=1, keepdims=True)
    bg_ref[...] = jnp.minimum(bg, float(E - 1)).astype(jnp.int32)
    total = jnp.sum(pc)
    valid_ref[...] = (brow < total).astype(jnp.int32)


def _router(xf, Wr, br2):
    return pl.pallas_call(
        _router_body,
        out_shape=(
            jax.ShapeDtypeStruct((S, 1), jnp.int32),
            jax.ShapeDtypeStruct((T, 16), jnp.float32),
            jax.ShapeDtypeStruct((T, 16), jnp.float32),
            jax.ShapeDtypeStruct((NB, 1), jnp.int32),
            jax.ShapeDtypeStruct((NB, 1), jnp.int32),
        ),
        scratch_shapes=[
            pltpu.VMEM((S, E), jnp.float32),
            pltpu.VMEM((S, 1), jnp.float32),
        ],
    )(xf, Wr, br2)


@functools.cache
def _sc_kernels():
    mesh = plsc.VectorSubcoreMesh(core_axis_name="c", subcore_axis_name="s")

    @functools.partial(
        pl.kernel,
        out_type=jax.ShapeDtypeStruct((RPAD, D), jnp.float32),
        mesh=mesh,
        scratch_types=[
            pltpu.VMEM((_NDCH, _DCH), jnp.int32),
            pltpu.VMEM((PPW, D), jnp.float32),
        ] + [pltpu.SemaphoreType.DMA] * (_NDCH + 2),
    )
    def _dispatch(x_hbm, dest_hbm, xs_hbm, dest_v, rows_v, *sems):
        wid = lax.axis_index("s") * SC_CORES + lax.axis_index("c")
        base = wid * PPW
        tok = lax.rem(base, T)
        dget = []
        rget = []
        for k in range(_NDCH):
            dget.append(pltpu.async_copy(
                dest_hbm.at[pl.ds(base + k * _DCH, _DCH)], dest_v.at[k],
                sems[_NDCH]))
            rget.append(pltpu.async_copy(
                x_hbm.at[pl.ds(tok + k * _DCH, _DCH)],
                rows_v.at[pl.ds(k * _DCH, _DCH), :], sems[k]))
        for c in dget:
            c.wait()
        scats = []
        for k in range(_NDCH):
            rget[k].wait()
            scats.append(pltpu.async_copy(
                rows_v.at[pl.ds(k * _DCH, _DCH), :],
                xs_hbm.at[dest_v.at[k]], sems[_NDCH + 1]))
        for c in scats:
            c.wait()

    @functools.partial(
        pl.kernel,
        out_type=jax.ShapeDtypeStruct((T, D), jnp.float32),
        mesh=mesh,
        scratch_types=[
            pltpu.VMEM((TPW,), jnp.int32),
            pltpu.VMEM((TPW,), jnp.int32),
            pltpu.VMEM((TPW, 16), jnp.float32),
            pltpu.VMEM((TPW, 16), jnp.float32),
            pltpu.VMEM((TPW, D), jnp.float32),
            pltpu.VMEM((TPW, D), jnp.float32),
        ] + [pltpu.SemaphoreType.DMA] * (2 * _NCH + 1),
    )
    def _combine(ys_hbm, dest_hbm, w0_hbm, w1_hbm, out_hbm,
                 p0_v, p1_v, w0_v, w1_v, r0_v, r1_v, *sems):
        wid = lax.axis_index("s") * SC_CORES + lax.axis_index("c")
        base = wid * TPW
        pltpu.sync_copy(dest_hbm.at[pl.ds(base, TPW)], p0_v)
        pltpu.sync_copy(dest_hbm.at[pl.ds(T + base, TPW)], p1_v)
        pltpu.sync_copy(w0_hbm.at[pl.ds(base, TPW)], w0_v)
        pltpu.sync_copy(w1_hbm.at[pl.ds(base, TPW)], w1_v)
        g0 = []
        g1 = []
        for k in range(_NCH):
            sl = pl.ds(k * _CH, _CH)
            g0.append(pltpu.async_copy(
                ys_hbm.at[p0_v.at[sl]], r0_v.at[sl, :], sems[k]))
            g1.append(pltpu.async_copy(
                ys_hbm.at[p1_v.at[sl]], r1_v.at[sl, :], sems[_NCH + k]))
        stores = []
        for k in range(_NCH):
            g0[k].wait()
            g1[k].wait()

            def row(i, _):
                a = w0_v[i, :]
                b = w1_v[i, :]
                for j in range(D // 16):
                    lj = pl.ds(j * 16, 16)
                    r0_v[i, lj] = a * r0_v[i, lj] + b * r1_v[i, lj]
                return 0

            lax.fori_loop(k * _CH, (k + 1) * _CH, row, 0)
            stores.append(pltpu.async_copy(
                r0_v.at[pl.ds(k * _CH, _CH), :],
                out_hbm.at[pl.ds(base + k * _CH, _CH)], sems[2 * _NCH]))
        for st in stores:
            st.wait()

    return _dispatch, _combine


def _ffn_body(bg_ref, valid_ref, xs_ref, w1_ref, b1_ref, w2_ref, b2_ref,
              ys_ref):
    w = pl.program_id(0)

    @pl.when(valid_ref[w] > 0)
    def _():
        x = xs_ref[...]
        h = lax.dot_general(x, w1_ref[0], (((1,), (1,)), ((), ())),
                            preferred_element_type=jnp.float32)
        h = h + b1_ref[0]
        h = 0.5 * h * (1.0 + lax.erf(h * 0.7071067811865476))
        y = lax.dot_general(h, w2_ref[0], (((1,), (1,)), ((), ())),
                            preferred_element_type=jnp.float32)
        ys_ref[...] = y + b2_ref[0]


def _ffn(bg, valid, xs, W1, b1, W2, b2):
    grid_spec = pltpu.PrefetchScalarGridSpec(
        num_scalar_prefetch=2,
        grid=(NB,),
        in_specs=[
            pl.BlockSpec((BT, D), lambda w, bg, vd: (w, 0)),
            pl.BlockSpec((1, H, D), lambda w, bg, vd: (bg[w], 0, 0)),
            pl.BlockSpec((1, 1, H), lambda w, bg, vd: (bg[w], 0, 0)),
            pl.BlockSpec((1, D, H), lambda w, bg, vd: (bg[w], 0, 0)),
            pl.BlockSpec((1, 1, D), lambda w, bg, vd: (bg[w], 0, 0)),
        ],
        out_specs=pl.BlockSpec((BT, D), lambda w, bg, vd: (w, 0)),
    )
    return pl.pallas_call(
        _ffn_body,
        grid_spec=grid_spec,
        out_shape=jax.ShapeDtypeStruct((RPAD, D), jnp.float32),
    )(bg, valid, xs, W1, b1, W2, b2)


def kernel(x, Wr, br, W1, b1, W2, b2):
    B, Sq, _ = x.shape
    xf = x.reshape(T, D)
    dest2, w0c, w1c, bg2, valid2 = _router(xf, Wr, br.reshape(1, E))
    dest = dest2.reshape(S)
    dispatch, combine = _sc_kernels()
    xs = dispatch(xf, dest)
    ys = _ffn(bg2.reshape(NB), valid2.reshape(NB), xs,
              W1, b1.reshape(E, 1, H),
              W2, b2.reshape(E, 1, D))
    out = combine(ys, dest, w0c, w1c)
    return out.reshape(B, Sq, D)
